# Initial kernel scaffold; baseline (speedup 1.0000x reference)
import jax
import jax.numpy as jnp
from jax.experimental import pallas as pl


def kernel(h_p, h_k, last_x, W_o, W_c, lin_w, lin_b, bias, mutual_index_p, mutual_index_k):
    raise NotImplementedError("write your pallas kernel here")


# speedup vs baseline: 1.3620x; 1.3620x over previous
import functools

import jax
import jax.numpy as jnp
from jax import lax
from jax.experimental import pallas as pl
from jax.experimental.pallas import tpu as pltpu
from jax.experimental.pallas import tpu_sc as plsc

_NC = 2
_NS = 16
_NW = _NC * _NS

_C = 128
_IB = 2048

_BLK = 1000
_OBLK = 512
_STEP = 2 * _BLK + _OBLK


def _gather_and_winner(h_p, h_k, idxp, idxk, M):
    N, D = h_p.shape
    Mpad = idxp.shape[0]
    nchunks = 2 * (Mpad // _C)
    R = Mpad // _C

    nb_full = M // _IB
    tail = M - nb_full * _IB

    mesh = plsc.VectorSubcoreMesh(core_axis_name="c", subcore_axis_name="s")

    @functools.partial(
        pl.kernel,
        mesh=mesh,
        out_type=[
            jax.ShapeDtypeStruct((Mpad, D), jnp.float32),
            jax.ShapeDtypeStruct((Mpad, D), jnp.float32),
            jax.ShapeDtypeStruct((N,), jnp.int32),
            jax.ShapeDtypeStruct((N,), jnp.int32),
        ],
        scratch_types=[
            pltpu.VMEM((N,), jnp.int32),
            pltpu.VMEM((_IB,), jnp.int32),
            pltpu.VMEM((_C, D), jnp.float32),
            pltpu.SemaphoreType.DMA,
        ],
        compiler_params=pltpu.CompilerParams(needs_layout_passes=False),
    )
    def k1(hp_ref, hk_ref, ip_ref, ik_ref, gp_ref, gk_ref, winp_ref, wink_ref,
           win, ibig, rows, sem):
        wid = lax.axis_index("s") * _NC + lax.axis_index("c")

        def group(idx16, mv):
            plsc.store_scatter(win, [idx16], mv)
            chk = plsc.load_gather(win, [idx16])

            def cond(c):
                return jnp.max(jnp.where(c < mv, 1, 0).astype(jnp.int32)) > 0

            def fix(c):
                plsc.store_scatter(win, [idx16], mv, mask=c < mv)
                return plsc.load_gather(win, [idx16])

            lax.while_loop(cond, fix, chk)

        def winner_work(idx_hbm, w_hbm):
            neg = jnp.full((16,), -1, jnp.int32)

            def init(q, _):
                win[pl.ds(q * 16, 16)] = neg
                return 0

            lax.fori_loop(0, N // 16, init, 0)

            def build_block(b, _):
                pltpu.sync_copy(idx_hbm.at[pl.ds(b * _IB, _IB)], ibig)

                def grp(q, _):
                    iv = ibig[pl.ds(q * 16, 16)]
                    mv = b * _IB + q * 16 + lax.iota(jnp.int32, 16)
                    group(iv, mv)
                    return 0

                lax.fori_loop(0, _IB // 16, grp, 0)
                return 0

            lax.fori_loop(0, nb_full, build_block, 0)
            if tail:
                pltpu.sync_copy(idx_hbm.at[pl.ds(nb_full * _IB, tail)],
                                ibig.at[pl.ds(0, tail)])

                def grp_t(q, _):
                    iv = ibig[pl.ds(q * 16, 16)]
                    mv = nb_full * _IB + q * 16 + lax.iota(jnp.int32, 16)
                    group(iv, mv)
                    return 0

                lax.fori_loop(0, tail // 16, grp_t, 0)

            pltpu.sync_copy(win, w_hbm)

        @pl.when(wid == 0)
        def _():
            winner_work(ip_ref, winp_ref)

        @pl.when(wid == 1)
        def _():
            winner_work(ik_ref, wink_ref)

        @pl.when(wid >= 2)
        def _():
            g = wid - 2
            n = (nchunks - g + (_NW - 3)) // (_NW - 2)

            def it(i, _):
                c = g + i * (_NW - 2)

                @pl.when(c < R)
                def _():
                    ib = ibig.at[pl.ds(0, _C)]
                    pltpu.sync_copy(ip_ref.at[pl.ds(c * _C, _C)], ib)
                    pltpu.async_copy(hp_ref.at[ib], rows, sem).wait()
                    pltpu.sync_copy(rows, gp_ref.at[pl.ds(c * _C, _C)])

                @pl.when(c >= R)
                def _():
                    c2 = c - R
                    ib = ibig.at[pl.ds(0, _C)]
                    pltpu.sync_copy(ik_ref.at[pl.ds(c2 * _C, _C)], ib)
                    pltpu.async_copy(hk_ref.at[ib], rows, sem).wait()
                    pltpu.sync_copy(rows, gk_ref.at[pl.ds(c2 * _C, _C)])

                return 0

            lax.fori_loop(0, n, it, 0)

    return k1(h_p, h_k, idxp, idxk)


def _dense(h_p, h_k, g_p, g_k, W_o, W_c, lwpT, lwkT, lin_b2, bias2):
    N, D = h_p.shape
    Mpad = g_p.shape[0]
    G = N // _BLK
    assert Mpad == G * _OBLK

    def body(hp, hk, gp, gk, wo, wc, lp, lk, lb, bs, cat):
        cat[0:_BLK, :] = jnp.dot(hp[...], wo[...],
                                 preferred_element_type=jnp.float32)
        cat[_BLK:2 * _BLK, :] = jnp.dot(hk[...], wc[...],
                                        preferred_element_type=jnp.float32)
        tp = jnp.tanh(jnp.dot(gp[...], wo[...],
                              preferred_element_type=jnp.float32))
        tk = jnp.tanh(jnp.dot(gk[...], wc[...],
                              preferred_element_type=jnp.float32))
        y = (jnp.dot(tp, lp[...], preferred_element_type=jnp.float32)
             + jnp.dot(tk, lk[...], preferred_element_type=jnp.float32)
             + lb[...])
        cat[2 * _BLK:_STEP, :] = jnp.where(y >= 0, y, 0.01 * y) + bs[...]

    full = pl.BlockSpec((D, D), lambda i: (0, 0))
    vec = pl.BlockSpec((1, D), lambda i: (0, 0))
    return pl.pallas_call(
        body,
        grid=(G,),
        in_specs=[
            pl.BlockSpec((_BLK, D), lambda i: (i, 0)),
            pl.BlockSpec((_BLK, D), lambda i: (i, 0)),
            pl.BlockSpec((_OBLK, D), lambda i: (i, 0)),
            pl.BlockSpec((_OBLK, D), lambda i: (i, 0)),
            full, full, full, full, vec, vec,
        ],
        out_specs=pl.BlockSpec((_STEP, D), lambda i: (i, 0)),
        out_shape=jax.ShapeDtypeStruct((G * _STEP, D), jnp.float32),
        compiler_params=pltpu.CompilerParams(
            dimension_semantics=("arbitrary",),
        ),
    )(h_p, h_k, g_p, g_k, W_o, W_c, lwpT, lwkT, lin_b2, bias2)


def _merge(cat, win_p, win_k, N, D):
    nfull = N // _C
    rem = N - nfull * _C
    npa = nfull + (1 if rem else 0)
    ntot = 2 * npa

    mesh = plsc.VectorSubcoreMesh(core_axis_name="c", subcore_axis_name="s")

    @functools.partial(
        pl.kernel,
        mesh=mesh,
        out_type=[
            jax.ShapeDtypeStruct((N, D), jnp.float32),
            jax.ShapeDtypeStruct((N, D), jnp.float32),
        ],
        scratch_types=[
            pltpu.VMEM((_C,), jnp.int32),
            pltpu.VMEM((_C,), jnp.int32),
            pltpu.VMEM((_C, D), jnp.float32),
            pltpu.SemaphoreType.DMA,
        ],
        compiler_params=pltpu.CompilerParams(needs_layout_passes=False),
    )
    def k2(cat_ref, winp_ref, wink_ref, outp_ref, outk_ref,
           wbuf, sbuf, rows, sem):
        wid = lax.axis_index("s") * _NC + lax.axis_index("c")
        n = (ntot - wid + _NW - 1) // _NW

        def do_chunk(win_hbm, out_hbm, arr_off, j, size):
            base = j * _C
            pltpu.sync_copy(win_hbm.at[pl.ds(base, size)],
                            wbuf.at[pl.ds(0, size)])
            lanes = lax.iota(jnp.int32, 16)

            def grp(q, _):
                wv = wbuf[pl.ds(q * 16, 16)]
                ig = base + q * 16 + lanes
                pos_raw = (lax.div(ig, _BLK) * _STEP + lax.rem(ig, _BLK)
                           + arr_off)
                pos_out = (lax.div(wv, _OBLK) * _STEP + lax.rem(wv, _OBLK)
                           + 2 * _BLK)
                sbuf[pl.ds(q * 16, 16)] = jnp.where(wv >= 0, pos_out, pos_raw)
                return 0

            lax.fori_loop(0, size // 16, grp, 0)
            pltpu.async_copy(cat_ref.at[sbuf.at[pl.ds(0, size)]],
                             rows.at[pl.ds(0, size)], sem).wait()
            pltpu.sync_copy(rows.at[pl.ds(0, size)],
                            out_hbm.at[pl.ds(base, size)])

        def it(i, _):
            c = wid + i * _NW

            @pl.when(c < npa)
            def _():
                @pl.when(c < nfull)
                def _():
                    do_chunk(winp_ref, outp_ref, 0, c, _C)

                if rem:
                    @pl.when(c == nfull)
                    def _():
                        do_chunk(winp_ref, outp_ref, 0, nfull, rem)

            @pl.when(c >= npa)
            def _():
                c2 = c - npa

                @pl.when(c2 < nfull)
                def _():
                    do_chunk(wink_ref, outk_ref, _BLK, c2, _C)

                if rem:
                    @pl.when(c2 == nfull)
                    def _():
                        do_chunk(wink_ref, outk_ref, _BLK, nfull, rem)

            return 0

        lax.fori_loop(0, n, it, 0)

    return k2(cat, win_p, win_k)


def kernel(h_p, h_k, last_x, W_o, W_c, lin_w, lin_b, bias,
           mutual_index_p, mutual_index_k):
    N, D = h_p.shape
    M = mutual_index_p.shape[0]

    G = N // _BLK
    Mpad = G * _OBLK
    pad = Mpad - M

    idxp = jnp.concatenate([mutual_index_p, mutual_index_p[:pad]])
    idxk = jnp.concatenate([mutual_index_k, mutual_index_k[:pad]])

    g_p, g_k, win_p, win_k = _gather_and_winner(h_p, h_k, idxp, idxk, M)

    lwpT = lin_w[:, :D].T
    lwkT = lin_w[:, D:].T
    cat = _dense(h_p, h_k, g_p, g_k, W_o, W_c, lwpT, lwkT,
                 lin_b.reshape(1, D), bias.reshape(1, D))

    fhp, fhk = _merge(cat, win_p, win_k, N, D)
    return fhp, fhk


# speedup vs baseline: 1.9709x; 1.4471x over previous
import functools

import jax
import jax.numpy as jnp
from jax import lax
from jax.experimental import pallas as pl
from jax.experimental.pallas import tpu as pltpu
from jax.experimental.pallas import tpu_sc as plsc

_NC = 2
_NS = 16
_NW = _NC * _NS

_C = 128
_MC = 80
_IB = 2048
_NB = 4

_BLK = 1000
_OBLK = 512
_STEP = 2 * _BLK + _OBLK


def _gather(h_p, h_k, idxp, idxk):
    N, D = h_p.shape
    Mpad = idxp.shape[0]
    R = Mpad // _C
    halfw = _NW // 2
    per_w = R // halfw
    rounds = (per_w + _NB - 1) // _NB

    mesh = plsc.VectorSubcoreMesh(core_axis_name="c", subcore_axis_name="s")

    @functools.partial(
        pl.kernel,
        mesh=mesh,
        out_type=[
            jax.ShapeDtypeStruct((Mpad, D), jnp.float32),
            jax.ShapeDtypeStruct((Mpad, D), jnp.float32),
        ],
        scratch_types=(
            [pltpu.VMEM((_C,), jnp.int32) for _ in range(_NB)]
            + [pltpu.VMEM((_C, D), jnp.float32) for _ in range(_NB)]
            + [pltpu.SemaphoreType.DMA for _ in range(3 * _NB)]
        ),
    )
    def k(hp_ref, hk_ref, ip_ref, ik_ref, gp_ref, gk_ref, *scr):
        ibuf = scr[:_NB]
        rows = scr[_NB:2 * _NB]
        si = scr[2 * _NB:3 * _NB]
        sg = scr[3 * _NB:4 * _NB]
        sw = scr[4 * _NB:5 * _NB]

        wid = lax.axis_index("s") * _NC + lax.axis_index("c")
        lw = lax.rem(wid, halfw)
        is_p = wid < halfw

        def work(idx_hbm, src_hbm, g_hbm):
            def rnd_body(r, _):
                c0 = lw * per_w + r * _NB

                for b in range(_NB):
                    @pl.when(r * _NB + b < per_w)
                    def _(b=b):
                        pltpu.async_copy(
                            idx_hbm.at[pl.ds((c0 + b) * _C, _C)],
                            ibuf[b], si[b])

                for b in range(_NB):
                    @pl.when(r * _NB + b < per_w)
                    def _(b=b):
                        pltpu.make_async_copy(
                            idx_hbm.at[pl.ds((c0 + b) * _C, _C)],
                            ibuf[b], si[b]).wait()
                        pltpu.async_copy(src_hbm.at[ibuf[b]], rows[b], sg[b])

                for b in range(_NB):
                    @pl.when(r * _NB + b < per_w)
                    def _(b=b):
                        pltpu.make_async_copy(src_hbm.at[ibuf[b]], rows[b],
                                              sg[b]).wait()
                        pltpu.async_copy(
                            rows[b], g_hbm.at[pl.ds((c0 + b) * _C, _C)],
                            sw[b])

                for b in range(_NB):
                    @pl.when(r * _NB + b < per_w)
                    def _(b=b):
                        pltpu.make_async_copy(
                            rows[b], g_hbm.at[pl.ds((c0 + b) * _C, _C)],
                            sw[b]).wait()

                return 0

            lax.fori_loop(0, rounds, rnd_body, 0)

        @pl.when(is_p)
        def _():
            work(ip_ref, hp_ref, gp_ref)

        @pl.when(jnp.logical_not(is_p))
        def _():
            work(ik_ref, hk_ref, gk_ref)

    return k(h_p, h_k, idxp, idxk)


def _winner(idxp, idxk, M, N):
    nb_full = M // _IB
    tail = M - nb_full * _IB

    mesh = plsc.VectorSubcoreMesh(core_axis_name="c", subcore_axis_name="s")

    @functools.partial(
        pl.kernel,
        mesh=mesh,
        out_type=[
            jax.ShapeDtypeStruct((N,), jnp.int32),
            jax.ShapeDtypeStruct((N,), jnp.int32),
        ],
        scratch_types=[
            pltpu.VMEM((N,), jnp.int32),
            pltpu.VMEM((_IB,), jnp.int32),
        ],
        compiler_params=pltpu.CompilerParams(needs_layout_passes=False),
    )
    def k(ip_ref, ik_ref, winp_ref, wink_ref, win, ibig):
        wid = lax.axis_index("s") * _NC + lax.axis_index("c")

        def group(idx16, mv):
            plsc.store_scatter(win, [idx16], mv)
            chk = plsc.load_gather(win, [idx16])

            def cond(c):
                return jnp.max(jnp.where(c < mv, 1, 0).astype(jnp.int32)) > 0

            def fix(c):
                plsc.store_scatter(win, [idx16], mv, mask=c < mv)
                return plsc.load_gather(win, [idx16])

            lax.while_loop(cond, fix, chk)

        def winner_work(idx_hbm, w_hbm):
            neg = jnp.full((16,), -1, jnp.int32)

            def init(q, _):
                win[pl.ds(q * 16, 16)] = neg
                return 0

            lax.fori_loop(0, N // 16, init, 0)

            def build_block(b, _):
                pltpu.sync_copy(idx_hbm.at[pl.ds(b * _IB, _IB)], ibig)

                def grp(q, _):
                    iv = ibig[pl.ds(q * 16, 16)]
                    mv = b * _IB + q * 16 + lax.iota(jnp.int32, 16)
                    group(iv, mv)
                    return 0

                lax.fori_loop(0, _IB // 16, grp, 0)
                return 0

            lax.fori_loop(0, nb_full, build_block, 0)
            if tail:
                pltpu.sync_copy(idx_hbm.at[pl.ds(nb_full * _IB, tail)],
                                ibig.at[pl.ds(0, tail)])

                def grp_t(q, _):
                    iv = ibig[pl.ds(q * 16, 16)]
                    mv = nb_full * _IB + q * 16 + lax.iota(jnp.int32, 16)
                    group(iv, mv)
                    return 0

                lax.fori_loop(0, tail // 16, grp_t, 0)

            pltpu.sync_copy(win, w_hbm)

        @pl.when(wid == 0)
        def _():
            winner_work(ip_ref, winp_ref)

        @pl.when(wid == 1)
        def _():
            winner_work(ik_ref, wink_ref)

    return k(idxp, idxk)


def _dense(h_p, h_k, g_p, g_k, W_o, W_c, lwpT, lwkT, lin_b2, bias2):
    N, D = h_p.shape
    Mpad = g_p.shape[0]
    G = N // _BLK
    assert Mpad == G * _OBLK

    def body(hp, hk, gp, gk, wo, wc, lp, lk, lb, bs, cat):
        cat[0:_BLK, :] = jnp.dot(hp[...], wo[...],
                                 preferred_element_type=jnp.float32)
        cat[_BLK:2 * _BLK, :] = jnp.dot(hk[...], wc[...],
                                        preferred_element_type=jnp.float32)
        tp = jnp.tanh(jnp.dot(gp[...], wo[...],
                              preferred_element_type=jnp.float32))
        tk = jnp.tanh(jnp.dot(gk[...], wc[...],
                              preferred_element_type=jnp.float32))
        y = (jnp.dot(tp, lp[...], preferred_element_type=jnp.float32)
             + jnp.dot(tk, lk[...], preferred_element_type=jnp.float32)
             + lb[...])
        cat[2 * _BLK:_STEP, :] = jnp.where(y >= 0, y, 0.01 * y) + bs[...]

    full = pl.BlockSpec((D, D), lambda i: (0, 0))
    vec = pl.BlockSpec((1, D), lambda i: (0, 0))
    return pl.pallas_call(
        body,
        grid=(G,),
        in_specs=[
            pl.BlockSpec((_BLK, D), lambda i: (i, 0)),
            pl.BlockSpec((_BLK, D), lambda i: (i, 0)),
            pl.BlockSpec((_OBLK, D), lambda i: (i, 0)),
            pl.BlockSpec((_OBLK, D), lambda i: (i, 0)),
            full, full, full, full, vec, vec,
        ],
        out_specs=pl.BlockSpec((_STEP, D), lambda i: (i, 0)),
        out_shape=jax.ShapeDtypeStruct((G * _STEP, D), jnp.float32),
        compiler_params=pltpu.CompilerParams(
            dimension_semantics=("arbitrary",),
        ),
    )(h_p, h_k, g_p, g_k, W_o, W_c, lwpT, lwkT, lin_b2, bias2)


def _merge(cat, win_p, win_k, N, D):
    npa = N // _MC
    ntot = 2 * npa

    mesh = plsc.VectorSubcoreMesh(core_axis_name="c", subcore_axis_name="s")

    @functools.partial(
        pl.kernel,
        mesh=mesh,
        out_type=[
            jax.ShapeDtypeStruct((N, D), jnp.float32),
            jax.ShapeDtypeStruct((N, D), jnp.float32),
        ],
        scratch_types=(
            [pltpu.VMEM((_MC,), jnp.int32) for _ in range(_NB)]
            + [pltpu.VMEM((_MC,), jnp.int32) for _ in range(_NB)]
            + [pltpu.VMEM((_MC, D), jnp.float32) for _ in range(_NB)]
            + [pltpu.SemaphoreType.DMA for _ in range(3 * _NB)]
        ),
        compiler_params=pltpu.CompilerParams(needs_layout_passes=False),
    )
    def k(cat_ref, winp_ref, wink_ref, outp_ref, outk_ref, *scr):
        wbuf = scr[:_NB]
        sbuf = scr[_NB:2 * _NB]
        rows = scr[2 * _NB:3 * _NB]
        si = scr[3 * _NB:4 * _NB]
        sg = scr[4 * _NB:5 * _NB]
        sw = scr[5 * _NB:6 * _NB]

        wid = lax.axis_index("s") * _NC + lax.axis_index("c")
        n = (ntot - wid + _NW - 1) // _NW
        rounds = (ntot + _NW * _NB - 1) // (_NW * _NB)
        lanes = lax.iota(jnp.int32, 16)

        def win_of(c):
            return c < npa

        def rnd_body(r, _):
            cs = []
            for b in range(_NB):
                cs.append(wid + (r * _NB + b) * _NW)

            for b in range(_NB):
                c = cs[b]

                @pl.when(c < npa)
                def _(b=b, c=c):
                    pltpu.async_copy(winp_ref.at[pl.ds(c * _MC, _MC)],
                                     wbuf[b], si[b])

                @pl.when(jnp.logical_and(c >= npa, c < ntot))
                def _(b=b, c=c):
                    pltpu.async_copy(wink_ref.at[pl.ds((c - npa) * _MC, _MC)],
                                     wbuf[b], si[b])

            for b in range(_NB):
                c = cs[b]

                @pl.when(c < ntot)
                def _(b=b, c=c):
                    pltpu.make_async_copy(winp_ref.at[pl.ds(0, _MC)],
                                          wbuf[b], si[b]).wait()
                    j = jnp.where(c < npa, c, c - npa)
                    arr_off = jnp.where(c < npa, 0, _BLK)
                    base = j * _MC

                    def grp(q, _):
                        wv = wbuf[b][pl.ds(q * 16, 16)]
                        ig = base + q * 16 + lanes
                        pos_raw = (lax.div(ig, _BLK) * _STEP
                                   + lax.rem(ig, _BLK) + arr_off)
                        pos_out = (lax.div(wv, _OBLK) * _STEP
                                   + lax.rem(wv, _OBLK) + 2 * _BLK)
                        sbuf[b][pl.ds(q * 16, 16)] = jnp.where(
                            wv >= 0, pos_out, pos_raw)
                        return 0

                    lax.fori_loop(0, _MC // 16, grp, 0)
                    pltpu.async_copy(cat_ref.at[sbuf[b]], rows[b], sg[b])

            for b in range(_NB):
                c = cs[b]

                @pl.when(c < ntot)
                def _(b=b, c=c):
                    pltpu.make_async_copy(cat_ref.at[sbuf[b]], rows[b],
                                          sg[b]).wait()

                @pl.when(c < npa)
                def _(b=b, c=c):
                    pltpu.async_copy(rows[b],
                                     outp_ref.at[pl.ds(c * _MC, _MC)], sw[b])

                @pl.when(jnp.logical_and(c >= npa, c < ntot))
                def _(b=b, c=c):
                    pltpu.async_copy(rows[b],
                                     outk_ref.at[pl.ds((c - npa) * _MC, _MC)],
                                     sw[b])

            for b in range(_NB):
                c = cs[b]

                @pl.when(c < ntot)
                def _(b=b, c=c):
                    pltpu.make_async_copy(rows[b],
                                          outp_ref.at[pl.ds(0, _MC)],
                                          sw[b]).wait()

            return 0

        del n
        lax.fori_loop(0, rounds, rnd_body, 0)

    return k(cat, win_p, win_k)


def kernel(h_p, h_k, last_x, W_o, W_c, lin_w, lin_b, bias,
           mutual_index_p, mutual_index_k):
    N, D = h_p.shape
    M = mutual_index_p.shape[0]

    G = N // _BLK
    Mpad = G * _OBLK
    pad = Mpad - M

    idxp = jnp.concatenate([mutual_index_p, mutual_index_p[:pad]])
    idxk = jnp.concatenate([mutual_index_k, mutual_index_k[:pad]])

    g_p, g_k = _gather(h_p, h_k, idxp, idxk)
    win_p, win_k = _winner(idxp, idxk, M, N)

    lwpT = lin_w[:, :D].T
    lwkT = lin_w[:, D:].T
    cat = _dense(h_p, h_k, g_p, g_k, W_o, W_c, lwpT, lwkT,
                 lin_b.reshape(1, D), bias.reshape(1, D))

    fhp, fhk = _merge(cat, win_p, win_k, N, D)
    return fhp, fhk
